# Initial kernel scaffold; baseline (speedup 1.0000x reference)
import jax
import jax.numpy as jnp
from jax.experimental import pallas as pl


def kernel(ids, weights):
    raise NotImplementedError("write your pallas kernel here")


# speedup vs baseline: 2.7415x; 2.7415x over previous
import functools

import jax
import jax.numpy as jnp
from jax import lax
from jax.experimental import pallas as pl
from jax.experimental.pallas import tpu as pltpu
from jax.experimental.pallas import tpu_sc as plsc

HALF = 16
RPG = 2
K = 8


def _tree_sum(vals):
    while len(vals) > 1:
        nxt = [vals[i] + vals[i + 1] for i in range(0, len(vals) - 1, 2)]
        if len(vals) % 2:
            nxt.append(vals[-1])
        vals = nxt
    return vals[0]


def kernel(ids, weights):
    B, L = ids.shape
    V, D = weights.shape
    info = plsc.get_sparse_core_info()
    nw = info.num_cores * info.num_subcores
    rows_w = B // nw
    idx_per_g = RPG * L
    ng = rows_w // RPG
    ids2 = ids.reshape(B // RPG, idx_per_g)

    mesh = plsc.VectorSubcoreMesh(core_axis_name="c", subcore_axis_name="s")

    @functools.partial(
        pl.kernel,
        mesh=mesh,
        compiler_params=pltpu.CompilerParams(use_tc_tiling_on_sc=False),
        out_type=jax.ShapeDtypeStruct((B, D), jnp.float32),
        scratch_types=[
            pltpu.VMEM((ng, idx_per_g), jnp.int32),
            pltpu.VMEM((K, idx_per_g, D), jnp.float32),
            pltpu.VMEM((rows_w, D), jnp.float32),
        ] + [pltpu.SemaphoreType.DMA] * K,
    )
    def run(ids_hbm, tab_hbm, out_hbm, ids_v, buf_v, out_v, *sems):
        wid = lax.axis_index("s") * info.num_cores + lax.axis_index("c")
        gbase = wid * ng
        rbase = wid * rows_w
        pltpu.sync_copy(ids_hbm.at[pl.ds(gbase, ng)], ids_v)

        def fire(g, s):
            pltpu.async_copy(tab_hbm.at[ids_v.at[g]], buf_v.at[s], sems[s])

        def drain(g, s):
            pltpu.make_async_copy(
                tab_hbm.at[ids_v.at[g]], buf_v.at[s], sems[s]).wait()

        for s in range(K):
            fire(s, s)

        def body(i, carry):
            gs = i * K
            for s in range(K):
                g = gs + s
                drain(g, s)
                for r in range(RPG):
                    lo = _tree_sum([buf_v[s, r * L + l, pl.ds(0, HALF)]
                                    for l in range(L)])
                    hi = _tree_sum([buf_v[s, r * L + l, pl.ds(HALF, HALF)]
                                    for l in range(L)])
                    row = g * RPG + r
                    out_v[row, pl.ds(0, HALF)] = lo
                    out_v[row, pl.ds(HALF, HALF)] = hi

                @pl.when(g + K < ng)
                def _():
                    fire(g + K, s)
            return carry

        lax.fori_loop(0, ng // K, body, 0)
        pltpu.sync_copy(out_v, out_hbm.at[pl.ds(rbase, rows_w)])

    return run(ids2, weights)
